# Initial kernel scaffold; baseline (speedup 1.0000x reference)
import jax
import jax.numpy as jnp
from jax.experimental import pallas as pl


def kernel(k, v):
    raise NotImplementedError("write your pallas kernel here")


# speedup vs baseline: 28.0951x; 28.0951x over previous
import jax
import jax.numpy as jnp
from jax import lax
from jax.experimental import pallas as pl
from jax.experimental.pallas import tpu as pltpu

N = 256
N_OUT = 192
R = N - N_OUT
NEG = -3e38


def _body(k_ref, v_ref, out_ref, sizes_ref, s_ref):
    kb = k_ref[0]
    nrm = jnp.sqrt(jnp.sum(kb * kb, axis=1, keepdims=True))
    kn = kb / nrm
    s = lax.dot_general(kn, kn, (((1,), (1,)), ((), ())),
                        preferred_element_type=jnp.float32)

    row_i = lax.broadcasted_iota(jnp.int32, (N, N), 0)
    col_j = lax.broadcasted_iota(jnp.int32, (N, N), 1)
    s = jnp.where(col_j > row_i, s, NEG)
    s_ref[...] = s
    lin = row_i * N + col_j

    lane = lax.broadcasted_iota(jnp.int32, (1, N), 1)
    gid_r0 = lane
    gid_c0 = lax.broadcasted_iota(jnp.int32, (N, 1), 0)

    def step(_, carry):
        gid_r, gid_c = carry
        s = s_ref[...]
        m = jnp.max(s)
        linsel = jnp.min(jnp.where(s == m, lin, N * N))
        i0 = linsel // N
        j0 = linsel % N
        g_i = jnp.sum(jnp.where(lane == i0, gid_r, 0))
        g_j = jnp.sum(jnp.where(lane == j0, gid_r, 0))
        gid_r = jnp.where(gid_r == g_i, g_j, gid_r)
        gid_c = jnp.where(gid_c == g_i, g_j, gid_c)
        mem_j_r = gid_r == g_j
        mem_j_c = gid_c == g_j
        s_ref[...] = jnp.where(mem_j_c & mem_j_r, NEG, s)
        return gid_r, gid_c

    gid_r, gid_c = lax.fori_loop(0, R, step, (gid_r0, gid_c0))

    lab_r = lane
    lab_c = gid_c0
    onehot = (gid_c == lab_r).astype(jnp.float32)
    memb = (lab_c == gid_r).astype(jnp.float32)
    counts_c = jnp.sum(memb, axis=1, keepdims=True)
    alive_c = (counts_c > 0).astype(jnp.float32)
    tlow = (col_j <= row_i).astype(jnp.float32)
    rank_c = lax.dot_general(tlow, alive_c, (((1,), (0,)), ((), ())),
                             preferred_element_type=jnp.float32) - 1.0
    slot_r = lane.astype(jnp.float32)
    sel = jnp.where((rank_c == slot_r) & (alive_c > 0), 1.0, 0.0)
    g_mat = lax.dot_general(onehot, sel, (((1,), (0,)), ((), ())),
                            preferred_element_type=jnp.float32)
    sizes_c = lax.dot_general(sel, counts_c, (((0,), (0,)), ((), ())),
                              preferred_element_type=jnp.float32)
    out_full = lax.dot_general(g_mat, v_ref[0], (((0,), (0,)), ((), ())),
                               preferred_element_type=jnp.float32)
    out_ref[0] = out_full / jnp.maximum(sizes_c, 1.0)
    sizes_ref[0] = jnp.sum(sel * counts_c, axis=0, keepdims=True)


def _run(k, v, interpret=False):
    B, n, D = k.shape
    out_pad, sizes_pad = pl.pallas_call(
        _body,
        grid=(B,),
        in_specs=[
            pl.BlockSpec((1, n, D), lambda b: (b, 0, 0)),
            pl.BlockSpec((1, n, D), lambda b: (b, 0, 0)),
        ],
        out_specs=[
            pl.BlockSpec((1, n, D), lambda b: (b, 0, 0)),
            pl.BlockSpec((1, 1, n), lambda b: (b, 0, 0)),
        ],
        out_shape=[
            jax.ShapeDtypeStruct((B, n, D), jnp.float32),
            jax.ShapeDtypeStruct((B, 1, n), jnp.float32),
        ],
        scratch_shapes=[pltpu.VMEM((n, n), jnp.float32)],
        interpret=interpret,
    )(k, v)
    out = out_pad[:, :N_OUT]
    sizes = jnp.round(sizes_pad[:, 0, :N_OUT]).astype(jnp.int32)
    return out, sizes


@jax.jit
def kernel(k, v):
    return _run(k, v)
